# baseline (speedup 1.0000x reference)
import jax
import jax.numpy as jnp
from jax import lax
from jax.experimental import pallas as pl
from jax.experimental.pallas import tpu as pltpu
from jax.experimental.pallas import tpu_sc as plsc

N_NODES = 10000
D = 128
N_EDGES = 320000

NC = 2
NS = 16
NW = NC * NS

CHUNK = 128
NCH = 80
E_PAD = NW * NCH * CHUNK
SLAB = 624
REM = N_NODES - SLAB * NS


def _scale_chunk(rows, vals):
  dn = lax.GatherDimensionNumbers(offset_dims=(), collapsed_slice_dims=(0,),
                                  start_index_map=(0,))

  def group(g, carry):
    vals16 = vals[pl.ds(g * 16, 16)]
    for j in range(16):
      w = lax.gather(vals16, jnp.full((16, 1), j, jnp.int32), dn,
                     slice_sizes=(1,),
                     mode=lax.GatherScatterMode.PROMISE_IN_BOUNDS)
      e = g * 16 + j
      for cg in range(D // 16):
        sl = pl.ds(cg * 16, 16)
        rows[e, sl] = rows[e, sl] * w
    return carry

  lax.fori_loop(0, CHUNK // 16, group, None)


def _sc_body(src_hbm, dst_hbm, val_hbm, emb_hbm, p0_hbm, p1_hbm,
             acc, dst_sl, src0, src1, val0, val1, rows0, rows1,
             gsem0, gsem1, ssem0, ssem1, isem0, isem1):
  cid = lax.axis_index("c")
  sid = lax.axis_index("s")
  wid = cid * NS + sid
  rows = (rows0, rows1)
  srcb = (src0, src1)
  valb = (val0, val1)
  gsem = (gsem0, gsem1)
  ssem = (ssem0, ssem1)
  isem = (isem0, isem1)

  zeros16 = jnp.zeros((16,), jnp.float32)

  def zrow(r, carry):
    for g in range(D // 16):
      rows0[r, pl.ds(g * 16, 16)] = zeros16
    return carry

  lax.fori_loop(0, CHUNK, zrow, None)
  for k in range(SLAB // CHUNK):
    pltpu.sync_copy(rows0.at[pl.ds(0, CHUNK)],
                    acc.at[pl.ds(sid * SLAB + k * CHUNK, CHUNK)])
  tail = SLAB % CHUNK
  if tail:
    pltpu.sync_copy(rows0.at[pl.ds(0, tail)],
                    acc.at[pl.ds(sid * SLAB + SLAB - tail, tail)])

  @pl.when(sid == NS - 1)
  def _():
    pltpu.sync_copy(rows0.at[pl.ds(0, REM)], acc.at[pl.ds(SLAB * NS, REM)])

  pltpu.sync_copy(dst_hbm.at[wid], dst_sl)
  plsc.subcore_barrier()

  def issue_edge(c, b):
    pltpu.async_copy(src_hbm.at[wid].at[c], srcb[b], isem[b])
    pltpu.async_copy(val_hbm.at[wid].at[c], valb[b], isem[b])

  def wait_edge(b):
    pltpu.make_async_copy(src_hbm.at[wid].at[0], srcb[b], isem[b]).wait()
    pltpu.make_async_copy(val_hbm.at[wid].at[0], valb[b], isem[b]).wait()

  def start_gather(b):
    pltpu.async_copy(emb_hbm.at[srcb[b]], rows[b], gsem[b])

  def wait_gather(b):
    pltpu.make_async_copy(emb_hbm.at[srcb[b]], rows[b], gsem[b]).wait()

  def start_scatter(c, b):
    pltpu.async_copy(rows[b], acc.at[dst_sl.at[c]], ssem[b], add=True)

  def wait_scatter(b):
    pltpu.make_async_copy(rows[b], acc.at[dst_sl.at[0]], ssem[b]).wait()

  def step(c, p, first=False):
    q = 1 - p
    wait_gather(p)

    @pl.when(c <= NCH - 2)
    def _():
      wait_edge(q)
      if not first:
        wait_scatter(q)
      start_gather(q)

    _scale_chunk(rows[p], valb[p])

    @pl.when(c <= NCH - 3)
    def _():
      issue_edge(c + 2, p)

    start_scatter(c, p)

  pltpu.sync_copy(src_hbm.at[wid].at[0], src0)
  pltpu.sync_copy(val_hbm.at[wid].at[0], val0)
  start_gather(0)
  issue_edge(1, 1)

  step(0, 0, first=True)
  step(1, 1)

  def pipe(g, carry):
    step(2 * g, 0)
    step(2 * g + 1, 1)
    return carry

  lax.fori_loop(1, NCH // 2, pipe, None)
  wait_scatter(0)
  wait_scatter(1)
  plsc.subcore_barrier()

  row0 = sid * SLAB

  @pl.when(cid == 0)
  def _():
    pltpu.sync_copy(acc.at[pl.ds(row0, SLAB)], p0_hbm.at[pl.ds(row0, SLAB)])

    @pl.when(sid == NS - 1)
    def _():
      pltpu.sync_copy(acc.at[pl.ds(SLAB * NS, REM)],
                      p0_hbm.at[pl.ds(SLAB * NS, REM)])

  @pl.when(cid == 1)
  def _():
    pltpu.sync_copy(acc.at[pl.ds(row0, SLAB)], p1_hbm.at[pl.ds(row0, SLAB)])

    @pl.when(sid == NS - 1)
    def _():
      pltpu.sync_copy(acc.at[pl.ds(SLAB * NS, REM)],
                      p1_hbm.at[pl.ds(SLAB * NS, REM)])


_sc_spmm = pl.kernel(
    _sc_body,
    out_type=(jax.ShapeDtypeStruct((N_NODES, D), jnp.float32),
              jax.ShapeDtypeStruct((N_NODES, D), jnp.float32)),
    mesh=plsc.VectorSubcoreMesh(core_axis_name="c", subcore_axis_name="s",
                                num_cores=NC, num_subcores=NS),
    scratch_types=[
        pltpu.VMEM_SHARED((N_NODES, D), jnp.float32),
        pltpu.VMEM((NCH, CHUNK), jnp.int32),
        pltpu.VMEM((CHUNK,), jnp.int32),
        pltpu.VMEM((CHUNK,), jnp.int32),
        pltpu.VMEM((CHUNK,), jnp.float32),
        pltpu.VMEM((CHUNK,), jnp.float32),
        pltpu.VMEM((CHUNK, D), jnp.float32),
        pltpu.VMEM((CHUNK, D), jnp.float32),
        pltpu.SemaphoreType.DMA,
        pltpu.SemaphoreType.DMA,
        pltpu.SemaphoreType.DMA,
        pltpu.SemaphoreType.DMA,
        pltpu.SemaphoreType.DMA,
        pltpu.SemaphoreType.DMA,
    ],
)


def _add_body(a_ref, b_ref, o_ref):
  o_ref[...] = a_ref[...] + b_ref[...]


def _combine(p0, p1):
  return pl.pallas_call(
      _add_body,
      out_shape=jax.ShapeDtypeStruct((N_NODES, D), jnp.float32),
      grid=(10,),
      in_specs=[pl.BlockSpec((N_NODES // 10, D), lambda i: (i, 0))] * 2,
      out_specs=pl.BlockSpec((N_NODES // 10, D), lambda i: (i, 0)),
  )(p0, p1)


def kernel(edge_index, edge_values, embeds):
  pad = E_PAD - N_EDGES
  dst = jnp.pad(edge_index[0].astype(jnp.int32), (0, pad))
  src = jnp.pad(edge_index[1].astype(jnp.int32), (0, pad))
  vals = jnp.pad(edge_values.astype(jnp.float32), (0, pad))
  dst = dst.reshape(NW, NCH, CHUNK)
  src = src.reshape(NW, NCH, CHUNK)
  vals = vals.reshape(NW, NCH, CHUNK)
  p0, p1 = _sc_spmm(src, dst, vals, embeds)
  return _combine(p0, p1)


# speedup vs baseline: 1.0003x; 1.0003x over previous
import jax
import jax.numpy as jnp
from jax import lax
from jax.experimental import pallas as pl
from jax.experimental.pallas import tpu as pltpu
from jax.experimental.pallas import tpu_sc as plsc

N_NODES = 10000
D = 128
N_EDGES = 320000

NC = 2
NS = 16
NW = NC * NS

CHUNK = 128
NCH = 80
GSPLIT = 4
E_PAD = NW * NCH * CHUNK
SLAB = 624
REM = N_NODES - SLAB * NS


def _scale_chunk(rows, vals):
  dn = lax.GatherDimensionNumbers(offset_dims=(), collapsed_slice_dims=(0,),
                                  start_index_map=(0,))

  def group(g, carry):
    vals16 = vals[pl.ds(g * 16, 16)]
    for j in range(16):
      w = lax.gather(vals16, jnp.full((16, 1), j, jnp.int32), dn,
                     slice_sizes=(1,),
                     mode=lax.GatherScatterMode.PROMISE_IN_BOUNDS)
      e = g * 16 + j
      for cg in range(D // 16):
        sl = pl.ds(cg * 16, 16)
        rows[e, sl] = rows[e, sl] * w
    return carry

  lax.fori_loop(0, CHUNK // 16, group, None)


def _sc_body(src_hbm, dst_hbm, val_hbm, emb_hbm, p0_hbm, p1_hbm,
             acc, dst_sl, src0, src1, val0, val1, rows0, rows1,
             gsem0, gsem1, ssem0, ssem1, isem0, isem1):
  cid = lax.axis_index("c")
  sid = lax.axis_index("s")
  wid = cid * NS + sid
  rows = (rows0, rows1)
  srcb = (src0, src1)
  valb = (val0, val1)
  gsem = (gsem0, gsem1)
  ssem = (ssem0, ssem1)
  isem = (isem0, isem1)

  zeros16 = jnp.zeros((16,), jnp.float32)

  def zrow(r, carry):
    for g in range(D // 16):
      rows0[r, pl.ds(g * 16, 16)] = zeros16
    return carry

  lax.fori_loop(0, CHUNK, zrow, None)
  for k in range(SLAB // CHUNK):
    pltpu.sync_copy(rows0.at[pl.ds(0, CHUNK)],
                    acc.at[pl.ds(sid * SLAB + k * CHUNK, CHUNK)])
  tail = SLAB % CHUNK
  if tail:
    pltpu.sync_copy(rows0.at[pl.ds(0, tail)],
                    acc.at[pl.ds(sid * SLAB + SLAB - tail, tail)])

  @pl.when(sid == NS - 1)
  def _():
    pltpu.sync_copy(rows0.at[pl.ds(0, REM)], acc.at[pl.ds(SLAB * NS, REM)])

  pltpu.sync_copy(dst_hbm.at[wid], dst_sl)
  plsc.subcore_barrier()

  def issue_edge(c, b):
    pltpu.async_copy(src_hbm.at[wid].at[c], srcb[b], isem[b])
    pltpu.async_copy(val_hbm.at[wid].at[c], valb[b], isem[b])

  def wait_edge(b):
    pltpu.make_async_copy(src_hbm.at[wid].at[0], srcb[b], isem[b]).wait()
    pltpu.make_async_copy(val_hbm.at[wid].at[0], valb[b], isem[b]).wait()

  def start_gather(b):
    gh = CHUNK // GSPLIT
    for h in range(GSPLIT):
      pltpu.async_copy(emb_hbm.at[srcb[b].at[pl.ds(h * gh, gh)]],
                       rows[b].at[pl.ds(h * gh, gh)], gsem[b])

  def wait_gather(b):
    pltpu.make_async_copy(emb_hbm.at[srcb[b]], rows[b], gsem[b]).wait()

  def start_scatter(c, b):
    pltpu.async_copy(rows[b], acc.at[dst_sl.at[c]], ssem[b], add=True)

  def wait_scatter(b):
    pltpu.make_async_copy(rows[b], acc.at[dst_sl.at[0]], ssem[b]).wait()

  def step(c, p, first=False):
    q = 1 - p
    wait_gather(p)

    @pl.when(c <= NCH - 2)
    def _():
      wait_edge(q)
      if not first:
        wait_scatter(q)
      start_gather(q)

    _scale_chunk(rows[p], valb[p])

    @pl.when(c <= NCH - 3)
    def _():
      issue_edge(c + 2, p)

    start_scatter(c, p)

  pltpu.sync_copy(src_hbm.at[wid].at[0], src0)
  pltpu.sync_copy(val_hbm.at[wid].at[0], val0)
  start_gather(0)
  issue_edge(1, 1)

  step(0, 0, first=True)
  step(1, 1)

  def pipe(g, carry):
    step(2 * g, 0)
    step(2 * g + 1, 1)
    return carry

  lax.fori_loop(1, NCH // 2, pipe, None)
  wait_scatter(0)
  wait_scatter(1)
  plsc.subcore_barrier()

  row0 = sid * SLAB

  @pl.when(cid == 0)
  def _():
    pltpu.sync_copy(acc.at[pl.ds(row0, SLAB)], p0_hbm.at[pl.ds(row0, SLAB)])

    @pl.when(sid == NS - 1)
    def _():
      pltpu.sync_copy(acc.at[pl.ds(SLAB * NS, REM)],
                      p0_hbm.at[pl.ds(SLAB * NS, REM)])

  @pl.when(cid == 1)
  def _():
    pltpu.sync_copy(acc.at[pl.ds(row0, SLAB)], p1_hbm.at[pl.ds(row0, SLAB)])

    @pl.when(sid == NS - 1)
    def _():
      pltpu.sync_copy(acc.at[pl.ds(SLAB * NS, REM)],
                      p1_hbm.at[pl.ds(SLAB * NS, REM)])


_sc_spmm = pl.kernel(
    _sc_body,
    out_type=(jax.ShapeDtypeStruct((N_NODES, D), jnp.float32),
              jax.ShapeDtypeStruct((N_NODES, D), jnp.float32)),
    mesh=plsc.VectorSubcoreMesh(core_axis_name="c", subcore_axis_name="s",
                                num_cores=NC, num_subcores=NS),
    scratch_types=[
        pltpu.VMEM_SHARED((N_NODES, D), jnp.float32),
        pltpu.VMEM((NCH, CHUNK), jnp.int32),
        pltpu.VMEM((CHUNK,), jnp.int32),
        pltpu.VMEM((CHUNK,), jnp.int32),
        pltpu.VMEM((CHUNK,), jnp.float32),
        pltpu.VMEM((CHUNK,), jnp.float32),
        pltpu.VMEM((CHUNK, D), jnp.float32),
        pltpu.VMEM((CHUNK, D), jnp.float32),
        pltpu.SemaphoreType.DMA,
        pltpu.SemaphoreType.DMA,
        pltpu.SemaphoreType.DMA,
        pltpu.SemaphoreType.DMA,
        pltpu.SemaphoreType.DMA,
        pltpu.SemaphoreType.DMA,
    ],
)


def _add_body(a_ref, b_ref, o_ref):
  o_ref[...] = a_ref[...] + b_ref[...]


def _combine(p0, p1):
  return pl.pallas_call(
      _add_body,
      out_shape=jax.ShapeDtypeStruct((N_NODES, D), jnp.float32),
      grid=(10,),
      in_specs=[pl.BlockSpec((N_NODES // 10, D), lambda i: (i, 0))] * 2,
      out_specs=pl.BlockSpec((N_NODES // 10, D), lambda i: (i, 0)),
  )(p0, p1)


def kernel(edge_index, edge_values, embeds):
  pad = E_PAD - N_EDGES
  dst = jnp.pad(edge_index[0].astype(jnp.int32), (0, pad))
  src = jnp.pad(edge_index[1].astype(jnp.int32), (0, pad))
  vals = jnp.pad(edge_values.astype(jnp.float32), (0, pad))
  dst = dst.reshape(NW, NCH, CHUNK)
  src = src.reshape(NW, NCH, CHUNK)
  vals = vals.reshape(NW, NCH, CHUNK)
  p0, p1 = _sc_spmm(src, dst, vals, embeds)
  return _combine(p0, p1)


# speedup vs baseline: 1.6292x; 1.6288x over previous
import jax
import jax.numpy as jnp
from jax import lax
from jax.experimental import pallas as pl
from jax.experimental.pallas import tpu as pltpu
from jax.experimental.pallas import tpu_sc as plsc

N_NODES = 10000
D = 128
N_EDGES = 320000

NC = 2
NS = 16
NW = NC * NS

CHUNK = 80
EDGES_PER_TILE = N_EDGES // NW
NCH = EDGES_PER_TILE // CHUNK
SLAB = 624
REM = N_NODES - SLAB * NS


def _scale_chunk(rows, val_v):
  dn = lax.GatherDimensionNumbers(offset_dims=(), collapsed_slice_dims=(0,),
                                  start_index_map=(0,))

  def group(g, carry):
    vals16 = val_v[pl.ds(g * 16, 16)]
    for j in range(16):
      w = lax.gather(vals16, jnp.full((16, 1), j, jnp.int32), dn,
                     slice_sizes=(1,),
                     mode=lax.GatherScatterMode.PROMISE_IN_BOUNDS)
      e = g * 16 + j
      for cg in range(D // 16):
        sl = pl.ds(cg * 16, 16)
        rows[e, sl] = rows[e, sl] * w
    return carry

  lax.fori_loop(0, CHUNK // 16, group, None)


def _sc_body(src_hbm, dst_hbm, val_hbm, emb_hbm, p0_hbm, p1_hbm,
             acc, src0, src1, dst_v, val_v, rows0, rows1, gsem0, gsem1):
  cid = lax.axis_index("c")
  sid = lax.axis_index("s")
  wid = cid * NS + sid
  rows = (rows0, rows1)
  srcb = (src0, src1)
  gsem = (gsem0, gsem1)

  zeros16 = jnp.zeros((16,), jnp.float32)

  def zrow(r, carry):
    for g in range(D // 16):
      rows0[r, pl.ds(g * 16, 16)] = zeros16
    return carry

  lax.fori_loop(0, CHUNK, zrow, None)
  for k in range(SLAB // CHUNK):
    pltpu.sync_copy(rows0, acc.at[pl.ds(sid * SLAB + k * CHUNK, CHUNK)])
  tail = SLAB % CHUNK
  if tail:
    pltpu.sync_copy(rows0.at[pl.ds(0, tail)],
                    acc.at[pl.ds(sid * SLAB + SLAB - tail, tail)])

  @pl.when(sid == NS - 1)
  def _():
    pltpu.sync_copy(rows0.at[pl.ds(0, REM)], acc.at[pl.ds(SLAB * NS, REM)])

  plsc.subcore_barrier()

  def start_gather(c, p):
    base = wid * EDGES_PER_TILE + c * CHUNK
    pltpu.sync_copy(src_hbm.at[pl.ds(base, CHUNK)], srcb[p])
    pltpu.async_copy(emb_hbm.at[srcb[p]], rows[p], gsem[p])

  def wait_gather(p):
    pltpu.make_async_copy(emb_hbm.at[srcb[p]], rows[p], gsem[p]).wait()

  def step(c, p):
    base = wid * EDGES_PER_TILE + c * CHUNK
    wait_gather(p)

    @pl.when(c < NCH - 1)
    def _():
      start_gather(c + 1, 1 - p)

    pltpu.sync_copy(val_hbm.at[pl.ds(base, CHUNK)], val_v)
    _scale_chunk(rows[p], val_v)
    pltpu.sync_copy(dst_hbm.at[pl.ds(base, CHUNK)], dst_v)
    pltpu.sync_copy(rows[p], acc.at[dst_v], add=True)

  start_gather(0, 0)
  step(0, 0)

  def pipe(g, carry):
    step(2 * g + 1, 1)
    step(2 * g + 2, 0)
    return carry

  lax.fori_loop(0, (NCH - 1) // 2, pipe, None)
  plsc.subcore_barrier()

  row0 = sid * SLAB

  @pl.when(cid == 0)
  def _():
    pltpu.sync_copy(acc.at[pl.ds(row0, SLAB)], p0_hbm.at[pl.ds(row0, SLAB)])

    @pl.when(sid == NS - 1)
    def _():
      pltpu.sync_copy(acc.at[pl.ds(SLAB * NS, REM)],
                      p0_hbm.at[pl.ds(SLAB * NS, REM)])

  @pl.when(cid == 1)
  def _():
    pltpu.sync_copy(acc.at[pl.ds(row0, SLAB)], p1_hbm.at[pl.ds(row0, SLAB)])

    @pl.when(sid == NS - 1)
    def _():
      pltpu.sync_copy(acc.at[pl.ds(SLAB * NS, REM)],
                      p1_hbm.at[pl.ds(SLAB * NS, REM)])


_sc_spmm = pl.kernel(
    _sc_body,
    out_type=(jax.ShapeDtypeStruct((N_NODES, D), jnp.float32),
              jax.ShapeDtypeStruct((N_NODES, D), jnp.float32)),
    mesh=plsc.VectorSubcoreMesh(core_axis_name="c", subcore_axis_name="s",
                                num_cores=NC, num_subcores=NS),
    scratch_types=[
        pltpu.VMEM_SHARED((N_NODES, D), jnp.float32),
        pltpu.VMEM((CHUNK,), jnp.int32),
        pltpu.VMEM((CHUNK,), jnp.int32),
        pltpu.VMEM((CHUNK,), jnp.int32),
        pltpu.VMEM((CHUNK,), jnp.float32),
        pltpu.VMEM((CHUNK, D), jnp.float32),
        pltpu.VMEM((CHUNK, D), jnp.float32),
        pltpu.SemaphoreType.DMA,
        pltpu.SemaphoreType.DMA,
    ],
)


def _add_body(a_ref, b_ref, o_ref):
  o_ref[...] = a_ref[...] + b_ref[...]


def _combine(p0, p1):
  return pl.pallas_call(
      _add_body,
      out_shape=jax.ShapeDtypeStruct((N_NODES, D), jnp.float32),
      grid=(10,),
      in_specs=[pl.BlockSpec((N_NODES // 10, D), lambda i: (i, 0))] * 2,
      out_specs=pl.BlockSpec((N_NODES // 10, D), lambda i: (i, 0)),
  )(p0, p1)


def kernel(edge_index, edge_values, embeds):
  dst = edge_index[0].astype(jnp.int32)
  src = edge_index[1].astype(jnp.int32)
  vals = edge_values.astype(jnp.float32)
  p0, p1 = _sc_spmm(src, dst, vals, embeds)
  return _combine(p0, p1)


# speedup vs baseline: 2.6715x; 1.6397x over previous
import jax
import jax.numpy as jnp
from jax import lax
from jax.experimental import pallas as pl
from jax.experimental.pallas import tpu as pltpu
from jax.experimental.pallas import tpu_sc as plsc

N_NODES = 10000
D = 128
N_EDGES = 320000

NC = 2
NS = 16
NW = NC * NS

CHUNK = 80
EDGES_PER_TILE = N_EDGES // NW
NCH = EDGES_PER_TILE // CHUNK
SLAB = 624
REM = N_NODES - SLAB * NS


def _scale_chunk(rows, val_v):
  dn = lax.GatherDimensionNumbers(offset_dims=(), collapsed_slice_dims=(0,),
                                  start_index_map=(0,))

  def group(g, carry):
    vals16 = val_v[pl.ds(g * 16, 16)]
    for j in range(16):
      w = lax.gather(vals16, jnp.full((16, 1), j, jnp.int32), dn,
                     slice_sizes=(1,),
                     mode=lax.GatherScatterMode.PROMISE_IN_BOUNDS)
      e = g * 16 + j
      for cg in range(D // 16):
        sl = pl.ds(cg * 16, 16)
        rows[e, sl] = rows[e, sl] * w
    return carry

  lax.fori_loop(0, CHUNK // 16, group, None)


def _sc_body(src_hbm, dst_hbm, val_hbm, emb_hbm, p0_hbm, p1_hbm,
             acc, src0, src1, dst0, dst1, val0, val1, rows0, rows1,
             gsem0, gsem1, ssem0, ssem1, esem0, esem1, dsem0, dsem1):
  cid = lax.axis_index("c")
  sid = lax.axis_index("s")
  wid = cid * NS + sid
  rows = (rows0, rows1)
  srcb = (src0, src1)
  dstb = (dst0, dst1)
  valb = (val0, val1)
  gsem = (gsem0, gsem1)
  ssem = (ssem0, ssem1)
  esem = (esem0, esem1)
  dsem = (dsem0, dsem1)

  zeros16 = jnp.zeros((16,), jnp.float32)

  def zrow(r, carry):
    for g in range(D // 16):
      rows0[r, pl.ds(g * 16, 16)] = zeros16
    return carry

  lax.fori_loop(0, CHUNK, zrow, None)
  for k in range(SLAB // CHUNK):
    pltpu.sync_copy(rows0, acc.at[pl.ds(sid * SLAB + k * CHUNK, CHUNK)])
  tail = SLAB % CHUNK
  if tail:
    pltpu.sync_copy(rows0.at[pl.ds(0, tail)],
                    acc.at[pl.ds(sid * SLAB + SLAB - tail, tail)])

  @pl.when(sid == NS - 1)
  def _():
    pltpu.sync_copy(rows0.at[pl.ds(0, REM)], acc.at[pl.ds(SLAB * NS, REM)])

  plsc.subcore_barrier()

  def ebase(c):
    return wid * EDGES_PER_TILE + c * CHUNK

  def issue_sv(c, b):
    pltpu.async_copy(src_hbm.at[pl.ds(ebase(c), CHUNK)], srcb[b], esem[b])
    pltpu.async_copy(val_hbm.at[pl.ds(ebase(c), CHUNK)], valb[b], esem[b])

  def wait_sv(b):
    pltpu.make_async_copy(src_hbm.at[pl.ds(0, CHUNK)], srcb[b],
                          esem[b]).wait()
    pltpu.make_async_copy(val_hbm.at[pl.ds(0, CHUNK)], valb[b],
                          esem[b]).wait()

  def issue_dst(c, b):
    pltpu.async_copy(dst_hbm.at[pl.ds(ebase(c), CHUNK)], dstb[b], dsem[b])

  def wait_dst(b):
    pltpu.make_async_copy(dst_hbm.at[pl.ds(0, CHUNK)], dstb[b],
                          dsem[b]).wait()

  def start_gather(b):
    pltpu.async_copy(emb_hbm.at[srcb[b]], rows[b], gsem[b])

  def wait_gather(b):
    pltpu.make_async_copy(emb_hbm.at[srcb[b]], rows[b], gsem[b]).wait()

  def start_scatter(b):
    pltpu.async_copy(rows[b], acc.at[dstb[b]], ssem[b], add=True)

  def wait_scatter(b):
    pltpu.make_async_copy(rows[b], acc.at[dstb[b]], ssem[b]).wait()

  def step(c, p, first=False):
    q = 1 - p
    wait_gather(p)

    @pl.when(c <= NCH - 2)
    def _():
      wait_sv(q)
      if not first:
        wait_scatter(q)
      start_gather(q)
      issue_dst(c + 1, q)

    _scale_chunk(rows[p], valb[p])

    @pl.when(c <= NCH - 3)
    def _():
      issue_sv(c + 2, p)

    if not first:
      wait_dst(p)
    start_scatter(p)

  pltpu.sync_copy(src_hbm.at[pl.ds(ebase(0), CHUNK)], src0)
  pltpu.sync_copy(val_hbm.at[pl.ds(ebase(0), CHUNK)], val0)
  pltpu.sync_copy(dst_hbm.at[pl.ds(ebase(0), CHUNK)], dst0)
  start_gather(0)
  issue_sv(1, 1)

  step(0, 0, first=True)

  def pipe(g, carry):
    step(2 * g + 1, 1)
    step(2 * g + 2, 0)
    return carry

  lax.fori_loop(0, (NCH - 1) // 2, pipe, None)
  wait_scatter(1)
  wait_scatter(0)
  plsc.subcore_barrier()

  row0 = sid * SLAB

  @pl.when(cid == 0)
  def _():
    pltpu.sync_copy(acc.at[pl.ds(row0, SLAB)], p0_hbm.at[pl.ds(row0, SLAB)])

    @pl.when(sid == NS - 1)
    def _():
      pltpu.sync_copy(acc.at[pl.ds(SLAB * NS, REM)],
                      p0_hbm.at[pl.ds(SLAB * NS, REM)])

  @pl.when(cid == 1)
  def _():
    pltpu.sync_copy(acc.at[pl.ds(row0, SLAB)], p1_hbm.at[pl.ds(row0, SLAB)])

    @pl.when(sid == NS - 1)
    def _():
      pltpu.sync_copy(acc.at[pl.ds(SLAB * NS, REM)],
                      p1_hbm.at[pl.ds(SLAB * NS, REM)])


_sc_spmm = pl.kernel(
    _sc_body,
    out_type=(jax.ShapeDtypeStruct((N_NODES, D), jnp.float32),
              jax.ShapeDtypeStruct((N_NODES, D), jnp.float32)),
    mesh=plsc.VectorSubcoreMesh(core_axis_name="c", subcore_axis_name="s",
                                num_cores=NC, num_subcores=NS),
    scratch_types=[
        pltpu.VMEM_SHARED((N_NODES, D), jnp.float32),
        pltpu.VMEM((CHUNK,), jnp.int32),
        pltpu.VMEM((CHUNK,), jnp.int32),
        pltpu.VMEM((CHUNK,), jnp.int32),
        pltpu.VMEM((CHUNK,), jnp.int32),
        pltpu.VMEM((CHUNK,), jnp.float32),
        pltpu.VMEM((CHUNK,), jnp.float32),
        pltpu.VMEM((CHUNK, D), jnp.float32),
        pltpu.VMEM((CHUNK, D), jnp.float32),
    ] + [pltpu.SemaphoreType.DMA] * 8,
)


def _add_body(a_ref, b_ref, o_ref):
  o_ref[...] = a_ref[...] + b_ref[...]


def _combine(p0, p1):
  return pl.pallas_call(
      _add_body,
      out_shape=jax.ShapeDtypeStruct((N_NODES, D), jnp.float32),
      grid=(10,),
      in_specs=[pl.BlockSpec((N_NODES // 10, D), lambda i: (i, 0))] * 2,
      out_specs=pl.BlockSpec((N_NODES // 10, D), lambda i: (i, 0)),
  )(p0, p1)


def kernel(edge_index, edge_values, embeds):
  dst = edge_index[0].astype(jnp.int32)
  src = edge_index[1].astype(jnp.int32)
  vals = edge_values.astype(jnp.float32)
  p0, p1 = _sc_spmm(src, dst, vals, embeds)
  return _combine(p0, p1)


# speedup vs baseline: 2.9843x; 1.1171x over previous
import jax
import jax.numpy as jnp
from jax import lax
from jax.experimental import pallas as pl
from jax.experimental.pallas import tpu as pltpu
from jax.experimental.pallas import tpu_sc as plsc

N_NODES = 10000
D = 128
N_EDGES = 320000

NC = 2
NS = 16
NW = NC * NS

CHUNK = 80
EDGES_PER_TILE = N_EDGES // NW
NCH = EDGES_PER_TILE // CHUNK
SLAB = 624
REM = N_NODES - SLAB * NS


def _scale_chunk(rows, val_v):
  dn = lax.GatherDimensionNumbers(offset_dims=(), collapsed_slice_dims=(0,),
                                  start_index_map=(0,))

  def group(g, carry):
    vals16 = val_v[pl.ds(g * 16, 16)]
    for j in range(16):
      w = lax.gather(vals16, jnp.full((16, 1), j, jnp.int32), dn,
                     slice_sizes=(1,),
                     mode=lax.GatherScatterMode.PROMISE_IN_BOUNDS)
      e = g * 16 + j
      for cg in range(D // 16):
        sl = pl.ds(cg * 16, 16)
        rows[e, sl] = rows[e, sl] * w
    return carry

  lax.fori_loop(0, CHUNK // 16, group, None)


def _sc_body(src_hbm, dst_hbm, val_hbm, emb_hbm, p0_hbm, p1_hbm,
             acc, src0, src1, src2, dst0, dst1, dst2, val0, val1, val2,
             rows0, rows1, rows2,
             gsem0, gsem1, gsem2, ssem0, ssem1, ssem2,
             esem0, esem1, esem2, dsem0, dsem1, dsem2):
  cid = lax.axis_index("c")
  sid = lax.axis_index("s")
  wid = cid * NS + sid
  rows = (rows0, rows1, rows2)
  srcb = (src0, src1, src2)
  dstb = (dst0, dst1, dst2)
  valb = (val0, val1, val2)
  gsem = (gsem0, gsem1, gsem2)
  ssem = (ssem0, ssem1, ssem2)
  esem = (esem0, esem1, esem2)
  dsem = (dsem0, dsem1, dsem2)

  zeros16 = jnp.zeros((16,), jnp.float32)

  def zrow(r, carry):
    for g in range(D // 16):
      rows0[r, pl.ds(g * 16, 16)] = zeros16
    return carry

  lax.fori_loop(0, CHUNK, zrow, None)
  for k in range(SLAB // CHUNK):
    pltpu.sync_copy(rows0, acc.at[pl.ds(sid * SLAB + k * CHUNK, CHUNK)])
  tail = SLAB % CHUNK
  if tail:
    pltpu.sync_copy(rows0.at[pl.ds(0, tail)],
                    acc.at[pl.ds(sid * SLAB + SLAB - tail, tail)])

  @pl.when(sid == NS - 1)
  def _():
    pltpu.sync_copy(rows0.at[pl.ds(0, REM)], acc.at[pl.ds(SLAB * NS, REM)])

  plsc.subcore_barrier()

  def ebase(c):
    return wid * EDGES_PER_TILE + c * CHUNK

  def issue_sv(c, b):
    pltpu.async_copy(src_hbm.at[pl.ds(ebase(c), CHUNK)], srcb[b], esem[b])
    pltpu.async_copy(val_hbm.at[pl.ds(ebase(c), CHUNK)], valb[b], esem[b])

  def wait_sv(b):
    pltpu.make_async_copy(src_hbm.at[pl.ds(0, CHUNK)], srcb[b],
                          esem[b]).wait()
    pltpu.make_async_copy(val_hbm.at[pl.ds(0, CHUNK)], valb[b],
                          esem[b]).wait()

  def issue_dst(c, b):
    pltpu.async_copy(dst_hbm.at[pl.ds(ebase(c), CHUNK)], dstb[b], dsem[b])

  def wait_dst(b):
    pltpu.make_async_copy(dst_hbm.at[pl.ds(0, CHUNK)], dstb[b],
                          dsem[b]).wait()

  def start_gather(b):
    pltpu.async_copy(emb_hbm.at[srcb[b]], rows[b], gsem[b])

  def wait_gather(b):
    pltpu.make_async_copy(emb_hbm.at[srcb[b]], rows[b], gsem[b]).wait()

  def start_scatter(b):
    pltpu.async_copy(rows[b], acc.at[dstb[b]], ssem[b], add=True)

  def wait_scatter(b):
    pltpu.make_async_copy(rows[b], acc.at[dstb[b]], ssem[b]).wait()

  def step(c, p, first=False):
    n2 = (p + 2) % 3
    wait_gather(p)

    @pl.when(c <= NCH - 3)
    def _():
      wait_sv(n2)
      if not first:
        wait_scatter(n2)
      start_gather(n2)
      issue_dst(c + 2, n2)

    _scale_chunk(rows[p], valb[p])

    @pl.when(c <= NCH - 4)
    def _():
      issue_sv(c + 3, p)

    if not first:
      wait_dst(p)
    start_scatter(p)

  pltpu.sync_copy(src_hbm.at[pl.ds(ebase(0), CHUNK)], src0)
  pltpu.sync_copy(val_hbm.at[pl.ds(ebase(0), CHUNK)], val0)
  pltpu.sync_copy(dst_hbm.at[pl.ds(ebase(0), CHUNK)], dst0)
  issue_sv(1, 1)
  issue_sv(2, 2)
  issue_dst(1, 1)
  start_gather(0)
  wait_sv(1)
  start_gather(1)

  step(0, 0, first=True)
  step(1, 1)

  def pipe(g, carry):
    step(3 * g + 2, 2)
    step(3 * g + 3, 0)
    step(3 * g + 4, 1)
    return carry

  lax.fori_loop(0, (NCH - 2) // 3, pipe, None)
  wait_scatter((NCH - 3) % 3)
  wait_scatter((NCH - 2) % 3)
  wait_scatter((NCH - 1) % 3)
  plsc.subcore_barrier()

  row0 = sid * SLAB

  @pl.when(cid == 0)
  def _():
    pltpu.sync_copy(acc.at[pl.ds(row0, SLAB)], p0_hbm.at[pl.ds(row0, SLAB)])

    @pl.when(sid == NS - 1)
    def _():
      pltpu.sync_copy(acc.at[pl.ds(SLAB * NS, REM)],
                      p0_hbm.at[pl.ds(SLAB * NS, REM)])

  @pl.when(cid == 1)
  def _():
    pltpu.sync_copy(acc.at[pl.ds(row0, SLAB)], p1_hbm.at[pl.ds(row0, SLAB)])

    @pl.when(sid == NS - 1)
    def _():
      pltpu.sync_copy(acc.at[pl.ds(SLAB * NS, REM)],
                      p1_hbm.at[pl.ds(SLAB * NS, REM)])


_sc_spmm = pl.kernel(
    _sc_body,
    out_type=(jax.ShapeDtypeStruct((N_NODES, D), jnp.float32),
              jax.ShapeDtypeStruct((N_NODES, D), jnp.float32)),
    mesh=plsc.VectorSubcoreMesh(core_axis_name="c", subcore_axis_name="s",
                                num_cores=NC, num_subcores=NS),
    scratch_types=[
        pltpu.VMEM_SHARED((N_NODES, D), jnp.float32),
        pltpu.VMEM((CHUNK,), jnp.int32),
        pltpu.VMEM((CHUNK,), jnp.int32),
        pltpu.VMEM((CHUNK,), jnp.int32),
        pltpu.VMEM((CHUNK,), jnp.int32),
        pltpu.VMEM((CHUNK,), jnp.int32),
        pltpu.VMEM((CHUNK,), jnp.int32),
        pltpu.VMEM((CHUNK,), jnp.float32),
        pltpu.VMEM((CHUNK,), jnp.float32),
        pltpu.VMEM((CHUNK,), jnp.float32),
        pltpu.VMEM((CHUNK, D), jnp.float32),
        pltpu.VMEM((CHUNK, D), jnp.float32),
        pltpu.VMEM((CHUNK, D), jnp.float32),
    ] + [pltpu.SemaphoreType.DMA] * 12,
)


def _add_body(a_ref, b_ref, o_ref):
  o_ref[...] = a_ref[...] + b_ref[...]


def _combine(p0, p1):
  return pl.pallas_call(
      _add_body,
      out_shape=jax.ShapeDtypeStruct((N_NODES, D), jnp.float32),
      grid=(10,),
      in_specs=[pl.BlockSpec((N_NODES // 10, D), lambda i: (i, 0))] * 2,
      out_specs=pl.BlockSpec((N_NODES // 10, D), lambda i: (i, 0)),
  )(p0, p1)


def kernel(edge_index, edge_values, embeds):
  ei = edge_index.astype(jnp.int32)
  p0, p1 = _sc_spmm(ei[1], ei[0], edge_values, embeds)
  return _combine(p0, p1)


# speedup vs baseline: 3.1862x; 1.0677x over previous
import jax
import jax.numpy as jnp
from jax import lax
from jax.experimental import pallas as pl
from jax.experimental.pallas import tpu as pltpu
from jax.experimental.pallas import tpu_sc as plsc

N_NODES = 10000
D = 128
N_EDGES = 320000

NC = 2
NS = 16
NW = NC * NS

CHUNK = 80
EDGES_PER_TILE = N_EDGES // NW
NCH = EDGES_PER_TILE // CHUNK
SLAB = 624
REM = N_NODES - SLAB * NS


def _scale_chunk(rows, val_v):
  dn = lax.GatherDimensionNumbers(offset_dims=(), collapsed_slice_dims=(0,),
                                  start_index_map=(0,))

  def group(g, carry):
    vals16 = val_v[pl.ds(g * 16, 16)]
    for j in range(16):
      w = lax.gather(vals16, jnp.full((16, 1), j, jnp.int32), dn,
                     slice_sizes=(1,),
                     mode=lax.GatherScatterMode.PROMISE_IN_BOUNDS)
      e = g * 16 + j
      for cg in range(D // 16):
        sl = pl.ds(cg * 16, 16)
        rows[e, sl] = rows[e, sl] * w
    return carry

  lax.fori_loop(0, CHUNK // 16, group, None)


def _sc_body(ei_hbm, val_hbm, emb_hbm, p0_hbm, p1_hbm,
             acc, src0, src1, src2, src3, dst0, dst1, dst2, dst3,
             val0, val1, val2, val3, rows0, rows1, rows2, rows3,
             gsem0, gsem1, gsem2, gsem3, ssem0, ssem1, ssem2, ssem3,
             esem0, esem1, esem2, esem3, dsem0, dsem1, dsem2, dsem3):
  cid = lax.axis_index("c")
  sid = lax.axis_index("s")
  wid = cid * NS + sid
  rows = (rows0, rows1, rows2, rows3)
  srcb = (src0, src1, src2, src3)
  dstb = (dst0, dst1, dst2, dst3)
  valb = (val0, val1, val2, val3)
  gsem = (gsem0, gsem1, gsem2, gsem3)
  ssem = (ssem0, ssem1, ssem2, ssem3)
  esem = (esem0, esem1, esem2, esem3)
  dsem = (dsem0, dsem1, dsem2, dsem3)

  zeros16 = jnp.zeros((16,), jnp.float32)

  def zrow(r, carry):
    for g in range(D // 16):
      rows0[r, pl.ds(g * 16, 16)] = zeros16
    return carry

  lax.fori_loop(0, CHUNK, zrow, None)
  for k in range(SLAB // CHUNK):
    pltpu.sync_copy(rows0, acc.at[pl.ds(sid * SLAB + k * CHUNK, CHUNK)])
  tail = SLAB % CHUNK
  if tail:
    pltpu.sync_copy(rows0.at[pl.ds(0, tail)],
                    acc.at[pl.ds(sid * SLAB + SLAB - tail, tail)])

  @pl.when(sid == NS - 1)
  def _():
    pltpu.sync_copy(rows0.at[pl.ds(0, REM)], acc.at[pl.ds(SLAB * NS, REM)])

  plsc.subcore_barrier()

  def ebase(c):
    return wid * EDGES_PER_TILE + c * CHUNK

  def issue_sv(c, b):
    pltpu.async_copy(ei_hbm.at[pl.ds(N_EDGES + ebase(c), CHUNK)], srcb[b],
                     esem[b])
    pltpu.async_copy(val_hbm.at[pl.ds(ebase(c), CHUNK)], valb[b], esem[b])

  def wait_sv(b):
    pltpu.make_async_copy(ei_hbm.at[pl.ds(0, CHUNK)], srcb[b],
                          esem[b]).wait()
    pltpu.make_async_copy(val_hbm.at[pl.ds(0, CHUNK)], valb[b],
                          esem[b]).wait()

  def issue_dst(c, b):
    pltpu.async_copy(ei_hbm.at[pl.ds(ebase(c), CHUNK)], dstb[b], dsem[b])

  def wait_dst(b):
    pltpu.make_async_copy(ei_hbm.at[pl.ds(0, CHUNK)], dstb[b],
                          dsem[b]).wait()

  def start_gather(b):
    pltpu.async_copy(emb_hbm.at[srcb[b]], rows[b], gsem[b])

  def wait_gather(b):
    pltpu.make_async_copy(emb_hbm.at[srcb[b]], rows[b], gsem[b]).wait()

  def start_scatter(b):
    pltpu.async_copy(rows[b], acc.at[dstb[b]], ssem[b], add=True)

  def wait_scatter(b):
    pltpu.make_async_copy(rows[b], acc.at[dstb[b]], ssem[b]).wait()

  def step(c, p, first=False):
    n3 = (p + 3) % 4
    wait_gather(p)

    @pl.when(c <= NCH - 4)
    def _():
      wait_sv(n3)
      if not first:
        wait_scatter(n3)
      start_gather(n3)
      issue_dst(c + 3, n3)

    _scale_chunk(rows[p], valb[p])

    @pl.when(c <= NCH - 5)
    def _():
      issue_sv(c + 4, p)

    if not first:
      wait_dst(p)
    start_scatter(p)

  pltpu.sync_copy(ei_hbm.at[pl.ds(N_EDGES + ebase(0), CHUNK)], src0)
  pltpu.sync_copy(val_hbm.at[pl.ds(ebase(0), CHUNK)], val0)
  pltpu.sync_copy(ei_hbm.at[pl.ds(ebase(0), CHUNK)], dst0)
  issue_sv(1, 1)
  issue_sv(2, 2)
  issue_sv(3, 3)
  issue_dst(1, 1)
  issue_dst(2, 2)
  start_gather(0)
  wait_sv(1)
  start_gather(1)
  wait_sv(2)
  start_gather(2)

  step(0, 0, first=True)
  step(1, 1)
  step(2, 2)

  def pipe(g, carry):
    step(4 * g + 3, 3)
    step(4 * g + 4, 0)
    step(4 * g + 5, 1)
    step(4 * g + 6, 2)
    return carry

  lax.fori_loop(0, (NCH - 3) // 4, pipe, None)
  step(NCH - 2, (NCH - 2) % 4)
  step(NCH - 1, (NCH - 1) % 4)
  wait_scatter((NCH - 4) % 4)
  wait_scatter((NCH - 3) % 4)
  wait_scatter((NCH - 2) % 4)
  wait_scatter((NCH - 1) % 4)
  plsc.subcore_barrier()

  row0 = sid * SLAB

  @pl.when(cid == 0)
  def _():
    pltpu.sync_copy(acc.at[pl.ds(row0, SLAB)], p0_hbm.at[pl.ds(row0, SLAB)])

    @pl.when(sid == NS - 1)
    def _():
      pltpu.sync_copy(acc.at[pl.ds(SLAB * NS, REM)],
                      p0_hbm.at[pl.ds(SLAB * NS, REM)])

  @pl.when(cid == 1)
  def _():
    pltpu.sync_copy(acc.at[pl.ds(row0, SLAB)], p1_hbm.at[pl.ds(row0, SLAB)])

    @pl.when(sid == NS - 1)
    def _():
      pltpu.sync_copy(acc.at[pl.ds(SLAB * NS, REM)],
                      p1_hbm.at[pl.ds(SLAB * NS, REM)])


_sc_spmm = pl.kernel(
    _sc_body,
    out_type=(jax.ShapeDtypeStruct((N_NODES, D), jnp.float32),
              jax.ShapeDtypeStruct((N_NODES, D), jnp.float32)),
    mesh=plsc.VectorSubcoreMesh(core_axis_name="c", subcore_axis_name="s",
                                num_cores=NC, num_subcores=NS),
    scratch_types=[
        pltpu.VMEM_SHARED((N_NODES, D), jnp.float32),
    ] + [pltpu.VMEM((CHUNK,), jnp.int32)] * 4
      + [pltpu.VMEM((CHUNK,), jnp.int32)] * 4
      + [pltpu.VMEM((CHUNK,), jnp.float32)] * 4
      + [pltpu.VMEM((CHUNK, D), jnp.float32)] * 4
      + [pltpu.SemaphoreType.DMA] * 16,
)


def _add_body(a_ref, b_ref, o_ref):
  o_ref[...] = a_ref[...] + b_ref[...]


def _combine(p0, p1):
  return pl.pallas_call(
      _add_body,
      out_shape=jax.ShapeDtypeStruct((N_NODES, D), jnp.float32),
      grid=(10,),
      in_specs=[pl.BlockSpec((N_NODES // 10, D), lambda i: (i, 0))] * 2,
      out_specs=pl.BlockSpec((N_NODES // 10, D), lambda i: (i, 0)),
  )(p0, p1)


def kernel(edge_index, edge_values, embeds):
  ei = edge_index.astype(jnp.int32).reshape(2 * N_EDGES)
  p0, p1 = _sc_spmm(ei, edge_values, embeds)
  return _combine(p0, p1)
